# baseline (speedup 1.0000x reference)
import functools

import jax
import jax.numpy as jnp
from jax import lax
from jax.experimental import pallas as pl
from jax.experimental.pallas import tpu as pltpu
from jax.experimental.pallas import tpu_sc as plsc

_INFO = plsc.get_sparse_core_info()
_NC = _INFO.num_cores
_NS = _INFO.num_subcores
_NW = _NC * _NS

_CHUNK = 512
_NBUF = 2


def _make_gather(n_rows: int, d: int):
    assert n_rows % _NW == 0
    rows_per_w = n_rows // _NW
    assert rows_per_w % (_CHUNK * _NBUF) == 0
    n_chunks = rows_per_w // _CHUNK
    n_outer = n_chunks // _NBUF

    mesh = plsc.VectorSubcoreMesh(core_axis_name="c", subcore_axis_name="s")

    @functools.partial(
        pl.kernel,
        mesh=mesh,
        out_type=jax.ShapeDtypeStruct((n_rows, d), jnp.float32),
        scratch_types=[
            pltpu.VMEM((rows_per_w,), jnp.int32),
            [pltpu.VMEM((_CHUNK, d), jnp.float32) for _ in range(_NBUF)],
            [pltpu.SemaphoreType.DMA for _ in range(_NBUF)],
            [pltpu.SemaphoreType.DMA for _ in range(_NBUF)],
        ],
        compiler_params=pltpu.CompilerParams(use_tc_tiling_on_sc=False),
    )
    def gather_kernel(table_hbm, idx_hbm, out_hbm, idx_v, bufs, gsems, ssems):
        wid = lax.axis_index("s") * _NC + lax.axis_index("c")
        base = wid * rows_per_w
        pltpu.sync_copy(idx_hbm.at[pl.ds(base, rows_per_w)], idx_v)

        def start_gather(g, b):
            pltpu.async_copy(
                table_hbm.at[idx_v.at[pl.ds(g * _CHUNK, _CHUNK)]],
                bufs[b], gsems[b],
            )

        def start_store(g, b):
            pltpu.async_copy(
                bufs[b], out_hbm.at[pl.ds(base + g * _CHUNK, _CHUNK)], ssems[b]
            )

        for b in range(_NBUF):
            start_gather(b, b)

        def body(go, carry):
            g0 = go * _NBUF
            for b in range(_NBUF):
                pltpu.make_async_copy(
                    table_hbm.at[pl.ds(0, _CHUNK)], bufs[b], gsems[b]
                ).wait()
                start_store(g0 + b, b)
            for b in range(_NBUF):
                pltpu.make_async_copy(
                    bufs[b], out_hbm.at[pl.ds(base, _CHUNK)], ssems[b]
                ).wait()
                start_gather(g0 + _NBUF + b, b)
            return carry

        lax.fori_loop(0, n_outer - 1, body, 0)

        g0 = (n_outer - 1) * _NBUF
        for b in range(_NBUF):
            pltpu.make_async_copy(
                table_hbm.at[pl.ds(0, _CHUNK)], bufs[b], gsems[b]
            ).wait()
            start_store(g0 + b, b)
        for b in range(_NBUF):
            pltpu.make_async_copy(
                bufs[b], out_hbm.at[pl.ds(base, _CHUNK)], ssems[b]
            ).wait()

    return gather_kernel


def kernel(data, ivectors):
    b, s = data.shape
    v, d = ivectors.shape
    idx = data.reshape(-1).astype(jnp.int32)
    out = _make_gather(b * s, d)(ivectors, idx)
    return out.reshape(b, s, d)


# speedup vs baseline: 1.0018x; 1.0018x over previous
import functools

import jax
import jax.numpy as jnp
from jax import lax
from jax.experimental import pallas as pl
from jax.experimental.pallas import tpu as pltpu
from jax.experimental.pallas import tpu_sc as plsc

_INFO = plsc.get_sparse_core_info()
_NC = _INFO.num_cores
_NS = _INFO.num_subcores
_NW = _NC * _NS

_CH_S = 8
_NBUF = 4


def _make_gather(bsz: int, seq: int, d: int):
    assert bsz % _NW == 0
    sent_per_w = bsz // _NW
    assert sent_per_w % (_CH_S * _NBUF) == 0
    n_chunks = sent_per_w // _CH_S
    n_outer = n_chunks // _NBUF
    rows = _CH_S * seq

    mesh = plsc.VectorSubcoreMesh(core_axis_name="c", subcore_axis_name="s")

    @functools.partial(
        pl.kernel,
        mesh=mesh,
        out_type=jax.ShapeDtypeStruct((bsz, seq, d), jnp.float32),
        scratch_types=[
            pltpu.VMEM((sent_per_w * seq,), jnp.int32),
            [pltpu.VMEM((rows, d), jnp.float32) for _ in range(_NBUF)],
            [pltpu.SemaphoreType.DMA for _ in range(_NBUF)],
            [pltpu.SemaphoreType.DMA for _ in range(_NBUF)],
        ],
        compiler_params=pltpu.CompilerParams(use_tc_tiling_on_sc=False),
    )
    def gather_kernel(table_hbm, idx_hbm, out_hbm, idx_v, bufs, gsems, ssems):
        wid = lax.axis_index("s") * _NC + lax.axis_index("c")
        base = wid * sent_per_w
        pltpu.sync_copy(
            idx_hbm.at[pl.ds(base * seq, sent_per_w * seq)], idx_v
        )

        def start_gather(g, b):
            pltpu.async_copy(
                table_hbm.at[idx_v.at[pl.ds(g * rows, rows)]],
                bufs[b], gsems[b],
            )

        def start_stores(g, b):
            for i in range(_CH_S):
                pltpu.async_copy(
                    bufs[b].at[pl.ds(i * seq, seq)],
                    out_hbm.at[base + g * _CH_S + i],
                    ssems[b],
                )

        def wait_gather(b):
            pltpu.make_async_copy(
                table_hbm.at[pl.ds(0, rows)], bufs[b], gsems[b]
            ).wait()

        def wait_stores(b):
            for _ in range(_CH_S):
                pltpu.make_async_copy(
                    bufs[b].at[pl.ds(0, seq)], out_hbm.at[0], ssems[b]
                ).wait()

        for b in range(_NBUF):
            start_gather(b, b)

        def body(go, carry):
            g0 = go * _NBUF
            for b in range(_NBUF):
                wait_gather(b)
                start_stores(g0 + b, b)
            for b in range(_NBUF):
                wait_stores(b)
                start_gather(g0 + _NBUF + b, b)
            return carry

        lax.fori_loop(0, n_outer - 1, body, 0)

        g0 = (n_outer - 1) * _NBUF
        for b in range(_NBUF):
            wait_gather(b)
            start_stores(g0 + b, b)
        for b in range(_NBUF):
            wait_stores(b)

    return gather_kernel


def kernel(data, ivectors):
    b, s = data.shape
    v, d = ivectors.shape
    idx = data.reshape(-1).astype(jnp.int32)
    return _make_gather(b, s, d)(ivectors, idx)


# speedup vs baseline: 1.0050x; 1.0032x over previous
import functools

import jax
import jax.numpy as jnp
from jax import lax
from jax.experimental import pallas as pl
from jax.experimental.pallas import tpu as pltpu
from jax.experimental.pallas import tpu_sc as plsc

_INFO = plsc.get_sparse_core_info()
_NC = _INFO.num_cores
_NS = _INFO.num_subcores
_NW = _NC * _NS

_CH_S = 8
_NBUF = 4


def _make_gather(bsz: int, seq: int, d: int):
    assert bsz % _NW == 0
    sent_per_w = bsz // _NW
    assert _NBUF == 4
    assert sent_per_w % (_CH_S * _NBUF) == 0
    n_chunks = sent_per_w // _CH_S
    n_outer = n_chunks // _NBUF
    assert n_outer >= 3
    rows = _CH_S * seq

    mesh = plsc.VectorSubcoreMesh(core_axis_name="c", subcore_axis_name="s")

    @functools.partial(
        pl.kernel,
        mesh=mesh,
        out_type=jax.ShapeDtypeStruct((bsz, seq, d), jnp.float32),
        scratch_types=[
            pltpu.VMEM((sent_per_w * seq,), jnp.int32),
            [pltpu.VMEM((rows, d), jnp.float32) for _ in range(_NBUF)],
            [pltpu.SemaphoreType.DMA for _ in range(_NBUF)],
            [pltpu.SemaphoreType.DMA for _ in range(_NBUF)],
        ],
        compiler_params=pltpu.CompilerParams(use_tc_tiling_on_sc=False),
    )
    def gather_kernel(table_hbm, idx_hbm, out_hbm, idx_v, bufs, gsems, ssems):
        wid = lax.axis_index("s") * _NC + lax.axis_index("c")
        base = wid * sent_per_w
        pltpu.sync_copy(
            idx_hbm.at[pl.ds(base * seq, sent_per_w * seq)], idx_v
        )

        def start_gather(g, b):
            pltpu.async_copy(
                table_hbm.at[idx_v.at[pl.ds(g * rows, rows)]],
                bufs[b], gsems[b],
            )

        def start_stores(g, b):
            for i in range(_CH_S):
                pltpu.async_copy(
                    bufs[b].at[pl.ds(i * seq, seq)],
                    out_hbm.at[base + g * _CH_S + i],
                    ssems[b],
                )

        def wait_gather(b):
            pltpu.make_async_copy(
                table_hbm.at[pl.ds(0, rows)], bufs[b], gsems[b]
            ).wait()

        def wait_stores(b):
            for _ in range(_CH_S):
                pltpu.make_async_copy(
                    bufs[b].at[pl.ds(0, seq)], out_hbm.at[0], ssems[b]
                ).wait()

        n = n_chunks
        start_gather(0, 0)
        start_gather(1, 1)
        wait_gather(0); start_stores(0, 0); start_gather(2, 2)
        wait_gather(1); start_stores(1, 1); start_gather(3, 3)
        wait_gather(2); start_stores(2, 2); wait_stores(0); start_gather(4, 0)
        wait_gather(3); start_stores(3, 3); wait_stores(1); start_gather(5, 1)

        def body(go, carry):
            c0 = (go + 1) * _NBUF
            for j in range(_NBUF):
                b2 = (j + 2) % _NBUF
                wait_gather(j)
                start_stores(c0 + j, j)
                wait_stores(b2)
                start_gather(c0 + j + 2, b2)
            return carry

        lax.fori_loop(0, n_outer - 2, body, 0)

        wait_gather(0); start_stores(n - 4, 0); wait_stores(2); start_gather(n - 2, 2)
        wait_gather(1); start_stores(n - 3, 1); wait_stores(3); start_gather(n - 1, 3)
        wait_gather(2); start_stores(n - 2, 2); wait_stores(0)
        wait_gather(3); start_stores(n - 1, 3); wait_stores(1)
        wait_stores(2)
        wait_stores(3)

    return gather_kernel


def kernel(data, ivectors):
    b, s = data.shape
    v, d = ivectors.shape
    idx = data.reshape(-1).astype(jnp.int32)
    return _make_gather(b, s, d)(ivectors, idx)
